# Initial kernel scaffold; baseline (speedup 1.0000x reference)
import jax
import jax.numpy as jnp
from jax.experimental import pallas as pl


def kernel(x, embeddings):
    raise NotImplementedError("write your pallas kernel here")


# speedup vs baseline: 3.4631x; 3.4631x over previous
import functools

import jax
import jax.numpy as jnp
from jax import lax
from jax.experimental import pallas as pl
from jax.experimental.pallas import tpu as pltpu
from jax.experimental.pallas import tpu_sc as plsc

_K = 8192
_D = 256
_LAYERS = 4

_BN = 256
_BK = 2048


def _dist_argmin_body(has_update, resid_ref, *rest):
    if has_update:
        upd_ref, emb_ref, nresid_ref, idx_ref, embsq_ref = rest
        r = resid_ref[...] - upd_ref[...]
        nresid_ref[...] = r
    else:
        emb_ref, idx_ref, embsq_ref = rest
        r = resid_ref[...]

    @pl.when(pl.program_id(0) == 0)
    def _():
        for c in range(_K // _BK):
            ec = emb_ref[:, c * _BK:(c + 1) * _BK]
            embsq_ref[:, c * _BK:(c + 1) * _BK] = jnp.sum(
                ec * ec, axis=0, keepdims=True)

    rsq = jnp.sum(r * r, axis=1, keepdims=True)
    best_v = None
    best_i = None
    for c in range(_K // _BK):
        ec = emb_ref[:, c * _BK:(c + 1) * _BK]
        sim = lax.dot_general(r, ec, (((1,), (0,)), ((), ())),
                              preferred_element_type=jnp.float32,
                              precision=lax.Precision.DEFAULT)
        d = (rsq + embsq_ref[:, c * _BK:(c + 1) * _BK]) - 2.0 * sim
        cmin = jnp.min(d, axis=1, keepdims=True)
        ii = lax.broadcasted_iota(jnp.int32, d.shape, 1) + c * _BK
        carg = jnp.min(jnp.where(d == cmin, ii, _K), axis=1, keepdims=True)
        if best_v is None:
            best_v, best_i = cmin, carg
        else:
            better = cmin < best_v
            best_v = jnp.where(better, cmin, best_v)
            best_i = jnp.where(better, carg, best_i)
    idx_ref[...] = jnp.reshape(best_i, (1, 1, _BN))


def _dist_argmin_call(resid, upd, emb, *, interpret=False):
    n = resid.shape[0]
    grid = (n // _BN,)
    row_spec = pl.BlockSpec((_BN, _D), lambda i: (i, 0))
    emb_spec = pl.BlockSpec((_D, _K), lambda i: (0, 0))
    idx_spec = pl.BlockSpec((1, 1, _BN), lambda i: (i, 0, 0))
    idx_shape = jax.ShapeDtypeStruct((n // _BN, 1, _BN), jnp.int32)
    scratch = [pltpu.VMEM((1, _K), jnp.float32)]
    if upd is None:
        return pl.pallas_call(
            functools.partial(_dist_argmin_body, False),
            grid=grid,
            in_specs=[row_spec, emb_spec],
            out_specs=idx_spec,
            out_shape=idx_shape,
            scratch_shapes=scratch,
            interpret=interpret,
        )(resid, emb)
    nresid, idx = pl.pallas_call(
        functools.partial(_dist_argmin_body, True),
        grid=grid,
        in_specs=[row_spec, row_spec, emb_spec],
        out_specs=[row_spec, idx_spec],
        out_shape=[jax.ShapeDtypeStruct((n, _D), jnp.float32), idx_shape],
        scratch_shapes=scratch,
        interpret=interpret,
    )(resid, upd, emb)
    return nresid, idx


def _combine_body(flat_ref, r_ref, upd_ref, out_ref):
    out_ref[...] = flat_ref[...] - r_ref[...] + upd_ref[...]


def _combine_call(flat, r, upd, *, interpret=False):
    n = flat.shape[0]
    bn = 1024
    spec = pl.BlockSpec((bn, _D), lambda i: (i, 0))
    return pl.pallas_call(
        _combine_body,
        grid=(n // bn,),
        in_specs=[spec, spec, spec],
        out_specs=spec,
        out_shape=jax.ShapeDtypeStruct((n, _D), jnp.float32),
        interpret=interpret,
    )(flat, r, upd)


@functools.cache
def _make_gather(n):
    info = plsc.get_sparse_core_info()
    nc, ns = info.num_cores, info.num_subcores
    nw = nc * ns
    b_per_w = n // nw
    ch = 128
    n_ch = b_per_w // ch
    mesh = plsc.VectorSubcoreMesh(core_axis_name="c", subcore_axis_name="s")

    @functools.partial(
        pl.kernel, mesh=mesh,
        out_type=jax.ShapeDtypeStruct((n, _D), jnp.float32),
        scratch_types=[
            pltpu.VMEM((ch,), jnp.int32),
            pltpu.VMEM((ch, _D), jnp.float32),
            pltpu.SemaphoreType.DMA,
        ],
    )
    def gather(table_hbm, idx_hbm, out_hbm, idx_v, rows_v, sem):
        wid = lax.axis_index("s") * nc + lax.axis_index("c")
        base = wid * b_per_w
        for c in range(n_ch):
            pltpu.sync_copy(idx_hbm.at[pl.ds(base + c * ch, ch)], idx_v)
            pltpu.async_copy(table_hbm.at[idx_v], rows_v, sem).wait()
            pltpu.sync_copy(rows_v, out_hbm.at[pl.ds(base + c * ch, ch)])

    return gather


def kernel(x, embeddings):
    shape = x.shape
    flat = x.reshape(-1, _D)
    n = flat.shape[0]
    emb_t = jnp.transpose(embeddings, (2, 1, 0)).astype(
        jnp.bfloat16).astype(jnp.float32)
    gather = _make_gather(n)

    idx = _dist_argmin_call(flat, None, embeddings[:, :, 0])
    upd = gather(emb_t[0], idx.reshape(-1))
    r = flat
    for layer in range(1, _LAYERS):
        r, idx = _dist_argmin_call(r, upd, embeddings[:, :, layer])
        upd = gather(emb_t[layer], idx.reshape(-1))
    out = _combine_call(flat, r, upd)
    return out.reshape(shape)
